# baseline (speedup 1.0000x reference)
import functools

import jax
import jax.numpy as jnp
from jax import lax
from jax.experimental import pallas as pl
from jax.experimental.pallas import tpu as pltpu
from jax.experimental.pallas import tpu_sc as plsc

VOCAB = 100000
EMB = 64
BATCH = 16384

TBLK = 2048


def _tblock(t_ref, o_ref):
    o_ref[...] = t_ref[...].T


@jax.jit
def _transpose_tc(table_t):
    grid = (VOCAB + TBLK - 1) // TBLK
    return pl.pallas_call(
        _tblock,
        grid=(grid,),
        in_specs=[pl.BlockSpec((EMB, TBLK), lambda j: (0, j))],
        out_specs=pl.BlockSpec((TBLK, EMB), lambda j: (j, 0)),
        out_shape=jax.ShapeDtypeStruct((VOCAB, EMB), jnp.float32),
    )(table_t)


@jax.jit
def _gather_sc(table, idx):
    info = plsc.get_sparse_core_info()
    nw = info.num_cores * info.num_subcores
    b_per_w = BATCH // nw
    n_tc = b_per_w // 128
    mesh = plsc.VectorSubcoreMesh(core_axis_name="c", subcore_axis_name="s")

    @functools.partial(
        pl.kernel,
        mesh=mesh,
        out_type=jax.ShapeDtypeStruct((EMB // 8, BATCH // 128, 8, 128), jnp.float32),
        scratch_types=[
            pltpu.VMEM((b_per_w,), jnp.int32),
            pltpu.VMEM((b_per_w, EMB), jnp.float32),
            pltpu.VMEM((EMB, b_per_w), jnp.float32),
            pltpu.SemaphoreType.DMA,
            pltpu.SemaphoreType.DMA,
        ],
        compiler_params=pltpu.CompilerParams(needs_layout_passes=False),
    )
    def k(table_hbm, idx_hbm, out4_hbm, idx_v, rows_v, out_t_v, sem, osem):
        wid = lax.axis_index("s") * info.num_cores + lax.axis_index("c")
        base = wid * b_per_w
        pltpu.sync_copy(idx_hbm.at[pl.ds(base, b_per_w)], idx_v)

        def body(c, _):
            vec = idx_v[pl.ds(c * 16, 16)]
            for j in range(16):
                r = vec[j]
                pltpu.async_copy(table_hbm.at[r], rows_v.at[c * 16 + j], sem)
            return 0

        lax.fori_loop(0, b_per_w // 16, body, 0)
        pltpu.make_async_copy(
            table_hbm.at[pl.ds(0, b_per_w)], rows_v, sem
        ).wait()

        iota16 = lax.iota(jnp.int32, 16)

        def tbody(i, _):
            col = jnp.full((16,), i, jnp.int32)
            for d0 in range(0, EMB, 16):
                vals = rows_v[i, pl.ds(d0, 16)]
                plsc.store_scatter(out_t_v, [d0 + iota16, col], vals)
            return 0

        lax.fori_loop(0, b_per_w, tbody, 0)

        for tcl in range(n_tc):
            for tr in range(EMB // 8):
                pltpu.async_copy(
                    out_t_v.at[pl.ds(tr * 8, 8), pl.ds(tcl * 128, 128)],
                    out4_hbm.at[tr, wid * n_tc + tcl],
                    osem,
                )
        for tcl in range(n_tc):
            for tr in range(EMB // 8):
                pltpu.make_async_copy(
                    out_t_v.at[pl.ds(0, 8), pl.ds(0, 128)],
                    out4_hbm.at[0, 0],
                    osem,
                ).wait()

    return k(table, idx)


def kernel(x, table):
    table_rm = _transpose_tc(table.T)
    out4 = _gather_sc(table_rm, x.astype(jnp.int32))
    return out4.transpose(0, 2, 1, 3).reshape(EMB, BATCH).T


# speedup vs baseline: 1.0938x; 1.0938x over previous
import functools

import jax
import jax.numpy as jnp
from jax import lax
from jax.experimental import pallas as pl
from jax.experimental.pallas import tpu as pltpu
from jax.experimental.pallas import tpu_sc as plsc

VOCAB = 100000
EMB = 64
BATCH = 16384

TBLK = 2048


def _tblock(t_ref, o_ref):
    o_ref[...] = t_ref[...].T


@jax.jit
def _transpose_tc(table_t):
    grid = (VOCAB + TBLK - 1) // TBLK
    return pl.pallas_call(
        _tblock,
        grid=(grid,),
        in_specs=[pl.BlockSpec((EMB, TBLK), lambda j: (0, j))],
        out_specs=pl.BlockSpec((TBLK, EMB), lambda j: (j, 0)),
        out_shape=jax.ShapeDtypeStruct((VOCAB, EMB), jnp.float32),
    )(table_t)


@jax.jit
def _gather_sc(table, idx):
    info = plsc.get_sparse_core_info()
    nw = info.num_cores * info.num_subcores
    b_per_w = BATCH // nw
    n_tc = b_per_w // 128
    mesh = plsc.VectorSubcoreMesh(core_axis_name="c", subcore_axis_name="s")

    @functools.partial(
        pl.kernel,
        mesh=mesh,
        out_type=jax.ShapeDtypeStruct((EMB // 8, BATCH // 128, 8, 128), jnp.float32),
        scratch_types=[
            pltpu.VMEM((b_per_w,), jnp.int32),
            pltpu.VMEM((b_per_w, EMB), jnp.float32),
            pltpu.VMEM((EMB, b_per_w), jnp.float32),
            pltpu.SemaphoreType.DMA,
            pltpu.SemaphoreType.DMA,
            pltpu.SemaphoreType.DMA,
        ],
        compiler_params=pltpu.CompilerParams(needs_layout_passes=False),
    )
    def k(table_hbm, idx_hbm, out4_hbm, idx_v, rows_v, out_t_v, sem_a, sem_b, osem):
        wid = lax.axis_index("s") * info.num_cores + lax.axis_index("c")
        base = wid * b_per_w
        pltpu.sync_copy(idx_hbm.at[pl.ds(base, b_per_w)], idx_v)

        iota16 = lax.iota(jnp.int32, 16)

        def fire(c, sem):
            vec = idx_v[pl.ds(c * 16, 16)]
            for j in range(16):
                r = vec[j]
                pltpu.async_copy(table_hbm.at[r], rows_v.at[c * 16 + j], sem)

        def drain_and_transpose(c, sem):
            pltpu.make_async_copy(
                table_hbm.at[pl.ds(0, 16)],
                rows_v.at[pl.ds(c * 16, 16)],
                sem,
            ).wait()
            rows = iota16 + c * 16
            for d in range(EMB):
                vals = plsc.load_gather(
                    rows_v, [rows, jnp.full((16,), d, jnp.int32)]
                )
                out_t_v[d, pl.ds(c * 16, 16)] = vals

        n_ch = b_per_w // 16
        fire(0, sem_a)
        fire(1, sem_b)

        def body(t, _):
            drain_and_transpose(2 * t, sem_a)
            fire(2 * t + 2, sem_a)
            drain_and_transpose(2 * t + 1, sem_b)
            fire(2 * t + 3, sem_b)
            return 0

        lax.fori_loop(0, n_ch // 2 - 1, body, 0)
        drain_and_transpose(n_ch - 2, sem_a)
        drain_and_transpose(n_ch - 1, sem_b)

        for tcl in range(n_tc):
            for tr in range(EMB // 8):
                pltpu.async_copy(
                    out_t_v.at[pl.ds(tr * 8, 8), pl.ds(tcl * 128, 128)],
                    out4_hbm.at[tr, wid * n_tc + tcl],
                    osem,
                )
        for tcl in range(n_tc):
            for tr in range(EMB // 8):
                pltpu.make_async_copy(
                    out_t_v.at[pl.ds(0, 8), pl.ds(0, 128)],
                    out4_hbm.at[0, 0],
                    osem,
                ).wait()

    return k(table, idx)


def kernel(x, table):
    out4 = _gather_sc(table, x.astype(jnp.int32))
    return out4.transpose(0, 2, 1, 3).reshape(EMB, BATCH).T
